# baseline (speedup 1.0000x reference)
import functools

import jax
import jax.numpy as jnp
from jax import lax
from jax.experimental import pallas as pl
from jax.experimental.pallas import tpu as pltpu
from jax.experimental.pallas import tpu_sc as plsc

B, D, T = 16, 256, 256
K = 8192
N = B * T
TK = 512
KT = K // TK


def _amin_body(z_ref, e_ref, idx_ref, loss_ref, best_ref, bidx_ref):
    k = pl.program_id(0)
    e = e_ref[...]
    esq = jnp.sum(e * e, axis=1, keepdims=True)
    for b in range(B):
        zb = z_ref[b]
        prod = lax.dot_general(
            e, zb, (((1,), (0,)), ((), ())),
            preferred_element_type=jnp.float32,
            precision=lax.Precision.DEFAULT)
        scores = esq - 2.0 * prod
        lmin = jnp.min(scores, axis=0, keepdims=True)
        rows = lax.broadcasted_iota(jnp.int32, (TK, T), 0)
        larg = jnp.min(jnp.where(scores == lmin, rows, TK),
                       axis=0, keepdims=True) + k * TK

        @pl.when(k == 0)
        def _():
            best_ref[b:b + 1, :] = lmin
            bidx_ref[b:b + 1, :] = larg

        @pl.when(k > 0)
        def _():
            prev = best_ref[b:b + 1, :]
            m = lmin < prev
            best_ref[b:b + 1, :] = jnp.where(m, lmin, prev)
            bidx_ref[b:b + 1, :] = jnp.where(m, larg, bidx_ref[b:b + 1, :])

    @pl.when(k == KT - 1)
    def _():
        idx_ref[...] = bidx_ref[...]
        tot = jnp.sum(best_ref[...])
        for b in range(B):
            zb = z_ref[b]
            tot = tot + jnp.sum(zb * zb)
        loss_ref[0, 0] = tot * (1.0 / (N * D))


_amin_call = pl.pallas_call(
    _amin_body,
    grid=(KT,),
    in_specs=[
        pl.BlockSpec((B, D, T), lambda k: (0, 0, 0)),
        pl.BlockSpec((TK, D), lambda k: (k, 0)),
    ],
    out_specs=[
        pl.BlockSpec((B, T), lambda k: (0, 0)),
        pl.BlockSpec((1, 1), lambda k: (0, 0), memory_space=pltpu.SMEM),
    ],
    out_shape=[
        jax.ShapeDtypeStruct((B, T), jnp.int32),
        jax.ShapeDtypeStruct((1, 1), jnp.float32),
    ],
    scratch_shapes=[
        pltpu.VMEM((B, T), jnp.float32),
        pltpu.VMEM((B, T), jnp.int32),
    ],
)


def _sc_call(idx, embedding):
    info = plsc.get_sparse_core_info()
    nc, ns = info.num_cores, info.num_subcores
    nw = nc * ns
    rpw = N // nw
    mesh = plsc.VectorSubcoreMesh(core_axis_name="c", subcore_axis_name="s")

    @functools.partial(
        pl.kernel,
        mesh=mesh,
        out_type=[
            jax.ShapeDtypeStruct((N, D), jnp.float32),
            jax.ShapeDtypeStruct((nw, K), jnp.int32),
        ],
        scratch_types=[
            pltpu.VMEM((rpw + 16,), jnp.int32),
            pltpu.VMEM((rpw, D), jnp.float32),
            pltpu.VMEM((K + 16,), jnp.int32),
            pltpu.SemaphoreType.DMA,
        ],
    )
    def body(idx_hbm, emb_hbm, zvq_hbm, cnt_hbm, idx_v, rows_v, cnt_v, sem):
        wid = lax.axis_index("s") * nc + lax.axis_index("c")
        base = wid * rpw
        pltpu.sync_copy(idx_hbm.at[pl.ds(base, rpw)], idx_v.at[pl.ds(0, rpw)])
        cp = pltpu.async_copy(emb_hbm.at[idx_v.at[pl.ds(0, rpw)]], rows_v, sem)

        def zero(i, _):
            cnt_v[pl.ds(i * 16, 16)] = jnp.zeros((16,), jnp.int32)
            return ()
        lax.fori_loop(0, (K + 16) // 16, zero, ())

        def hist(i, _):
            c = idx_v[pl.ds(i, 16)][0]
            one_hot0 = jnp.maximum(1 - lax.iota(jnp.int32, 16), 0)
            cnt_v[pl.ds(c, 16)] = cnt_v[pl.ds(c, 16)] + one_hot0
            return ()
        lax.fori_loop(0, rpw, hist, ())

        cp.wait()
        pltpu.sync_copy(rows_v, zvq_hbm.at[pl.ds(base, rpw)])
        pltpu.sync_copy(cnt_v.at[pl.ds(0, K)], cnt_hbm.at[wid])

    return body(idx, embedding)


def _fin_body(zvq_ref, cnt_ref, out_ref, ppl_ref):
    b = pl.program_id(0)
    out_ref[0] = jnp.transpose(zvq_ref[0])

    @pl.when(b == 0)
    def _():
        c = jnp.sum(cnt_ref[...], axis=0, keepdims=True)
        p = c.astype(jnp.float32) * (1.0 / N)
        ent = jnp.sum(p * jnp.log(p + 1e-10))
        ppl_ref[0, 0] = jnp.exp(-ent)


def _fin_call(zvq3, cnts):
    nw = cnts.shape[0]
    return pl.pallas_call(
        _fin_body,
        grid=(B,),
        in_specs=[
            pl.BlockSpec((1, T, D), lambda b: (b, 0, 0)),
            pl.BlockSpec((nw, K), lambda b: (0, 0)),
        ],
        out_specs=[
            pl.BlockSpec((1, D, T), lambda b: (b, 0, 0)),
            pl.BlockSpec((1, 1), lambda b: (0, 0), memory_space=pltpu.SMEM),
        ],
        out_shape=[
            jax.ShapeDtypeStruct((B, D, T), jnp.float32),
            jax.ShapeDtypeStruct((1, 1), jnp.float32),
        ],
    )(zvq3, cnts)


def kernel(z, embedding):
    idx2d, loss11 = _amin_call(z, embedding)
    zvq, cnts = _sc_call(idx2d.reshape(N), embedding)
    zout, ppl11 = _fin_call(zvq.reshape(B, T, D), cnts)
    loss = loss11[0, 0]
    return zout, loss, loss, ppl11[0, 0]


# speedup vs baseline: 1.9303x; 1.9303x over previous
import functools

import jax
import jax.numpy as jnp
from jax import lax
from jax.experimental import pallas as pl
from jax.experimental.pallas import tpu as pltpu
from jax.experimental.pallas import tpu_sc as plsc

B, D, T = 16, 256, 256
K = 8192
N = B * T
TK = 8192
KT = K // TK


CH = 128
NCH = TK // CH


def _amin_body(z_ref, e_ref, idx_ref, loss_ref, best_ref, bidx_ref):
    k = pl.program_id(0)
    e = e_ref[...]
    esq2 = jnp.sum(e * e, axis=1, keepdims=True) * 0.5
    rowsf = lax.broadcasted_iota(jnp.int32, (CH, T), 0).astype(jnp.float32)
    for b in range(B):
        zb = z_ref[b]
        lmax = None
        for c in range(NCH):
            ec = e_ref[c * CH:(c + 1) * CH, :]
            prod = lax.dot_general(
                ec, zb, (((1,), (0,)), ((), ())),
                preferred_element_type=jnp.float32,
                precision=lax.Precision.DEFAULT)
            h = prod - esq2[c * CH:(c + 1) * CH, :]
            cmax = jnp.max(h, axis=0, keepdims=True)
            cargf = jnp.min(jnp.where(h == cmax, rowsf, 3e7),
                            axis=0, keepdims=True)
            if lmax is None:
                lmax = cmax
                largf = cargf
            else:
                m = cmax > lmax
                lmax = jnp.where(m, cmax, lmax)
                largf = jnp.where(m, cargf + float(c * CH), largf)
        largf = largf + (k * TK).astype(jnp.float32)

        @pl.when(k == 0)
        def _():
            best_ref[b:b + 1, :] = lmax
            bidx_ref[b:b + 1, :] = largf

        @pl.when(k > 0)
        def _():
            prev = best_ref[b:b + 1, :]
            m = lmax > prev
            best_ref[b:b + 1, :] = jnp.where(m, lmax, prev)
            bidx_ref[b:b + 1, :] = jnp.where(m, largf, bidx_ref[b:b + 1, :])

    @pl.when(k == KT - 1)
    def _():
        idx_ref[...] = bidx_ref[...].astype(jnp.int32)
        tot = -2.0 * jnp.sum(best_ref[...])
        for b in range(B):
            zb = z_ref[b]
            tot = tot + jnp.sum(zb * zb)
        loss_ref[0, 0] = tot * (1.0 / (N * D))


_amin_call = pl.pallas_call(
    _amin_body,
    grid=(KT,),
    in_specs=[
        pl.BlockSpec((B, D, T), lambda k: (0, 0, 0)),
        pl.BlockSpec((TK, D), lambda k: (k, 0)),
    ],
    out_specs=[
        pl.BlockSpec((B, T), lambda k: (0, 0)),
        pl.BlockSpec((1, 1), lambda k: (0, 0), memory_space=pltpu.SMEM),
    ],
    out_shape=[
        jax.ShapeDtypeStruct((B, T), jnp.int32),
        jax.ShapeDtypeStruct((1, 1), jnp.float32),
    ],
    scratch_shapes=[
        pltpu.VMEM((B, T), jnp.float32),
        pltpu.VMEM((B, T), jnp.float32),
    ],
)


def _sc_call(idx, embedding):
    info = plsc.get_sparse_core_info()
    ns = info.num_subcores
    nw = ns
    nc = 1
    rpw = N // nw
    mesh = plsc.VectorSubcoreMesh(core_axis_name="c", subcore_axis_name="s", num_cores=1)

    @functools.partial(
        pl.kernel,
        mesh=mesh,
        out_type=[
            jax.ShapeDtypeStruct((N, D), jnp.float32),
            jax.ShapeDtypeStruct((nw, K), jnp.int32),
        ],
        scratch_types=[
            pltpu.VMEM((rpw + 16,), jnp.int32),
            pltpu.VMEM((rpw, D), jnp.float32),
            pltpu.VMEM((K + 16,), jnp.int32),
            pltpu.SemaphoreType.DMA,
        ],
    )
    def body(idx_hbm, emb_hbm, zvq_hbm, cnt_hbm, idx_v, rows_v, cnt_v, sem):
        wid = lax.axis_index("s") * nc + lax.axis_index("c")
        base = wid * rpw
        pltpu.sync_copy(idx_hbm.at[pl.ds(base, rpw)], idx_v.at[pl.ds(0, rpw)])
        cp = pltpu.async_copy(emb_hbm.at[idx_v.at[pl.ds(0, rpw)]], rows_v, sem)

        def zero(i, _):
            cnt_v[pl.ds(i * 16, 16)] = jnp.zeros((16,), jnp.int32)
            return ()
        lax.fori_loop(0, (K + 16) // 16, zero, (), unroll=16)

        def hist(i, _):
            c = idx_v[pl.ds(i, 16)][0]
            one_hot0 = jnp.maximum(1 - lax.iota(jnp.int32, 16), 0)
            plsc.addupdate(cnt_v.at[pl.ds(c, 16)], one_hot0)
            return ()
        lax.fori_loop(0, rpw, hist, (), unroll=8)

        cp.wait()
        pltpu.sync_copy(rows_v, zvq_hbm.at[pl.ds(base, rpw)])
        pltpu.sync_copy(cnt_v.at[pl.ds(0, K)], cnt_hbm.at[wid])

    return body(idx, embedding)


def _fin_body(zvq_ref, cnt_ref, out_ref, ppl_ref):
    for b in range(B):
        out_ref[b] = jnp.transpose(zvq_ref[b])
    c = jnp.sum(cnt_ref[...], axis=0, keepdims=True)
    p = c.astype(jnp.float32) * (1.0 / N)
    ent = jnp.sum(p * jnp.log(p + 1e-10))
    ppl_ref[0, 0] = jnp.exp(-ent)


def _fin_call(zvq3, cnts):
    nw = cnts.shape[0]
    return pl.pallas_call(
        _fin_body,
        grid=(1,),
        in_specs=[
            pl.BlockSpec((B, T, D), lambda i: (0, 0, 0)),
            pl.BlockSpec((nw, K), lambda i: (0, 0)),
        ],
        out_specs=[
            pl.BlockSpec((B, D, T), lambda i: (0, 0, 0)),
            pl.BlockSpec((1, 1), lambda i: (0, 0), memory_space=pltpu.SMEM),
        ],
        out_shape=[
            jax.ShapeDtypeStruct((B, D, T), jnp.float32),
            jax.ShapeDtypeStruct((1, 1), jnp.float32),
        ],
    )(zvq3, cnts)


def kernel(z, embedding):
    idx2d, loss11 = _amin_call(z, embedding)
    zvq, cnts = _sc_call(idx2d.reshape(N), embedding)
    zout, ppl11 = _fin_call(zvq.reshape(B, T, D), cnts)
    loss = loss11[0, 0]
    return zout, loss, loss, ppl11[0, 0]


# speedup vs baseline: 1.9365x; 1.0032x over previous
import functools

import jax
import jax.numpy as jnp
from jax import lax
from jax.experimental import pallas as pl
from jax.experimental.pallas import tpu as pltpu
from jax.experimental.pallas import tpu_sc as plsc

B, D, T = 16, 256, 256
K = 8192
N = B * T
TK = 8192
KT = K // TK


CH = 128
NCH = TK // CH


def _amin_body(z_ref, e_ref, idx_ref, loss_ref, best_ref, bidx_ref):
    k = pl.program_id(0)
    e = e_ref[...]
    esq2 = jnp.sum(e * e, axis=1, keepdims=True) * 0.5
    rowsf = lax.broadcasted_iota(jnp.int32, (CH, T), 0).astype(jnp.float32)
    for b in range(B):
        zb = z_ref[b]
        lmax = None
        for c in range(NCH):
            ec = e_ref[c * CH:(c + 1) * CH, :]
            prod = lax.dot_general(
                ec, zb, (((1,), (0,)), ((), ())),
                preferred_element_type=jnp.float32,
                precision=lax.Precision.DEFAULT)
            h = prod - esq2[c * CH:(c + 1) * CH, :]
            cmax = jnp.max(h, axis=0, keepdims=True)
            cargf = jnp.min(jnp.where(h == cmax, rowsf, 3e7),
                            axis=0, keepdims=True)
            if lmax is None:
                lmax = cmax
                largf = cargf
            else:
                m = cmax > lmax
                lmax = jnp.where(m, cmax, lmax)
                largf = jnp.where(m, cargf + float(c * CH), largf)
        largf = largf + (k * TK).astype(jnp.float32)

        @pl.when(k == 0)
        def _():
            best_ref[b:b + 1, :] = lmax
            bidx_ref[b:b + 1, :] = largf

        @pl.when(k > 0)
        def _():
            prev = best_ref[b:b + 1, :]
            m = lmax > prev
            best_ref[b:b + 1, :] = jnp.where(m, lmax, prev)
            bidx_ref[b:b + 1, :] = jnp.where(m, largf, bidx_ref[b:b + 1, :])

    @pl.when(k == KT - 1)
    def _():
        idx_ref[...] = bidx_ref[...].astype(jnp.int32)
        tot = -2.0 * jnp.sum(best_ref[...])
        for b in range(B):
            zb = z_ref[b]
            tot = tot + jnp.sum(zb * zb)
        loss_ref[0, 0] = tot * (1.0 / (N * D))


_amin_call = pl.pallas_call(
    _amin_body,
    grid=(KT,),
    in_specs=[
        pl.BlockSpec((B, D, T), lambda k: (0, 0, 0)),
        pl.BlockSpec((TK, D), lambda k: (k, 0)),
    ],
    out_specs=[
        pl.BlockSpec((B, T), lambda k: (0, 0)),
        pl.BlockSpec((1, 1), lambda k: (0, 0), memory_space=pltpu.SMEM),
    ],
    out_shape=[
        jax.ShapeDtypeStruct((B, T), jnp.int32),
        jax.ShapeDtypeStruct((1, 1), jnp.float32),
    ],
    scratch_shapes=[
        pltpu.VMEM((B, T), jnp.float32),
        pltpu.VMEM((B, T), jnp.float32),
    ],
)


def _sc_call(idx, embedding):
    info = plsc.get_sparse_core_info()
    ns = info.num_subcores
    nw = ns
    nc = 1
    rpw = N // nw
    mesh = plsc.VectorSubcoreMesh(core_axis_name="c", subcore_axis_name="s", num_cores=1)

    @functools.partial(
        pl.kernel,
        mesh=mesh,
        out_type=[
            jax.ShapeDtypeStruct((N, D), jnp.float32),
            jax.ShapeDtypeStruct((nw, K), jnp.int32),
        ],
        scratch_types=[
            pltpu.VMEM((rpw + 16,), jnp.int32),
            pltpu.VMEM((rpw, D), jnp.float32),
            pltpu.VMEM((K + 16,), jnp.int32),
            pltpu.SemaphoreType.DMA,
        ],
    )
    def body(idx_hbm, emb_hbm, zvq_hbm, cnt_hbm, idx_v, rows_v, cnt_v, sem):
        wid = lax.axis_index("s") * nc + lax.axis_index("c")
        base = wid * rpw
        pltpu.sync_copy(idx_hbm.at[pl.ds(base, rpw)], idx_v.at[pl.ds(0, rpw)])
        cp = pltpu.async_copy(emb_hbm.at[idx_v.at[pl.ds(0, rpw)]], rows_v, sem)

        def zero(i, _):
            cnt_v[pl.ds(i * 16, 16)] = jnp.zeros((16,), jnp.int32)
            return ()
        lax.fori_loop(0, (K + 16) // 16, zero, (), unroll=4)

        def hist(g, _):
            iv = idx_v[pl.ds(g * 16, 16)]
            one_hot0 = jnp.maximum(1 - lax.iota(jnp.int32, 16), 0)
            for l in range(16):
                plsc.addupdate(cnt_v.at[pl.ds(iv[l], 16)], one_hot0)
            return ()
        lax.fori_loop(0, rpw // 16, hist, ())

        cp.wait()
        pltpu.sync_copy(rows_v, zvq_hbm.at[pl.ds(base, rpw)])
        pltpu.sync_copy(cnt_v.at[pl.ds(0, K)], cnt_hbm.at[wid])

    return body(idx, embedding)


def _fin_body(zvq_ref, cnt_ref, out_ref, ppl_ref):
    for b in range(B):
        out_ref[b] = jnp.transpose(zvq_ref[b])
    c = jnp.sum(cnt_ref[...], axis=0, keepdims=True)
    p = c.astype(jnp.float32) * (1.0 / N)
    ent = jnp.sum(p * jnp.log(p + 1e-10))
    ppl_ref[0, 0] = jnp.exp(-ent)


def _fin_call(zvq3, cnts):
    nw = cnts.shape[0]
    return pl.pallas_call(
        _fin_body,
        grid=(1,),
        in_specs=[
            pl.BlockSpec((B, T, D), lambda i: (0, 0, 0)),
            pl.BlockSpec((nw, K), lambda i: (0, 0)),
        ],
        out_specs=[
            pl.BlockSpec((B, D, T), lambda i: (0, 0, 0)),
            pl.BlockSpec((1, 1), lambda i: (0, 0), memory_space=pltpu.SMEM),
        ],
        out_shape=[
            jax.ShapeDtypeStruct((B, D, T), jnp.float32),
            jax.ShapeDtypeStruct((1, 1), jnp.float32),
        ],
    )(zvq3, cnts)


def kernel(z, embedding):
    idx2d, loss11 = _amin_call(z, embedding)
    zvq, cnts = _sc_call(idx2d.reshape(N), embedding)
    zout, ppl11 = _fin_call(zvq.reshape(B, T, D), cnts)
    loss = loss11[0, 0]
    return zout, loss, loss, ppl11[0, 0]
